# baseline (speedup 1.0000x reference)
import jax
import jax.numpy as jnp
from jax import lax
from jax.experimental import pallas as pl
from jax.experimental.pallas import tpu as pltpu
from jax.experimental.pallas import tpu_sc as plsc

N = 2048
H = 12
HP = 16
NBUCKETS = 32
D = 2 * N
BI = 1024

_THRESHOLDS = (12, 16, 23, 32, 46, 64, 91)

_SC_WORKERS = 32
_SC_CHUNK = D // _SC_WORKERS
_SC_NC = 2


def _diag_sc_body(tableP_hbm, diagT_hbm, idx_v, rows_v, sem):
    c = lax.axis_index("c")
    s = lax.axis_index("s")
    base = (s * _SC_NC + c) * _SC_CHUNK
    for j in range(_SC_CHUNK // 16):
        p = base + j * 16 + lax.iota(jnp.int32, 16)
        d = p - (N - 1)
        nneg = -d
        na = jnp.abs(nneg)
        one = jnp.full((16,), 1, jnp.int32)
        zero = jnp.full((16,), 0, jnp.int32)
        large = jnp.full((16,), 8, jnp.int32)
        for t in _THRESHOLDS:
            large = large + jnp.where(na >= t, one, zero)
        bucket = jnp.where(na < 8, na, large) + jnp.where(
            nneg < 0, jnp.full((16,), 16, jnp.int32), zero
        )
        idx_v[pl.ds(j * 16, 16)] = bucket
    pltpu.async_copy(tableP_hbm.at[idx_v], rows_v, sem).wait()
    pltpu.sync_copy(rows_v, diagT_hbm.at[pl.ds(base, _SC_CHUNK), :])


_diag_sc = pl.kernel(
    _diag_sc_body,
    out_type=jax.ShapeDtypeStruct((D, HP), jnp.float32),
    mesh=plsc.VectorSubcoreMesh(core_axis_name="c", subcore_axis_name="s"),
    scratch_types=[
        pltpu.VMEM((_SC_CHUNK,), jnp.int32),
        pltpu.VMEM((_SC_CHUNK, HP), jnp.float32),
        pltpu.SemaphoreType.DMA,
    ],
    compiler_params=pltpu.CompilerParams(use_tc_tiling_on_sc=False),
)


def _expand_tc_body(diagT_ref, out_ref, rot_ref, diag_all_ref, sems):
    h = pl.program_id(0)

    @pl.when(h == 0)
    def _transpose_and_mask():
        dall = jnp.transpose(diagT_ref[...], (1, 0))
        p = jax.lax.broadcasted_iota(jnp.int32, (1, D), 1)
        d = p - (N - 1)
        mask = -(((d > 32) | (d < -32)).astype(jnp.float32) * 100000000.0)
        diag_all_ref[...] = dall + jnp.broadcast_to(mask, (HP, D))

    bc = jnp.broadcast_to(diag_all_ref[pl.ds(h, 1), :], (128, D))
    rot_ref[...] = pltpu.roll(bc, D - 127, 1, stride=1, stride_axis=0)

    copies = []
    for g in range(N // 128):
        a = pl.multiple_of((N - 1 - 127) - g * 128, 128)
        cp = pltpu.make_async_copy(
            rot_ref.at[:, pl.ds(a, N)],
            out_ref.at[h, pl.ds(g * 128, 128), :],
            sems.at[g],
        )
        cp.start()
        copies.append(cp)
    for cp in copies:
        cp.wait()


@jax.jit
def _bias_impl(table):
    tableP = jnp.pad(table, ((0, 0), (0, HP - H)))
    diagT = _diag_sc(tableP)
    return pl.pallas_call(
        _expand_tc_body,
        grid=(H,),
        in_specs=[pl.BlockSpec((D, HP), lambda h: (0, 0))],
        out_specs=pl.BlockSpec(memory_space=pl.ANY),
        out_shape=jax.ShapeDtypeStruct((H, N, N), jnp.float32),
        scratch_shapes=[
            pltpu.VMEM((128, D), jnp.float32),
            pltpu.VMEM((HP, D), jnp.float32),
            pltpu.SemaphoreType.DMA((N // 128,)),
        ],
        compiler_params=pltpu.CompilerParams(
            dimension_semantics=("arbitrary",),
        ),
    )(diagT)


def kernel(n, table):
    del n
    return _bias_impl(table)


# speedup vs baseline: 1.1412x; 1.1412x over previous
import jax
import jax.numpy as jnp
from jax import lax
from jax.experimental import pallas as pl
from jax.experimental.pallas import tpu as pltpu
from jax.experimental.pallas import tpu_sc as plsc

N = 2048
H = 12
HP = 16
NBUCKETS = 32
D = 2 * N
BI = 1024

_THRESHOLDS = (12, 16, 23, 32, 46, 64, 91)

_SC_WORKERS = 32
_SC_CHUNK = D // _SC_WORKERS
_SC_NC = 2


def _diag_sc_body(tableP_hbm, diagT_hbm, idx_v, rows_v, sem):
    c = lax.axis_index("c")
    s = lax.axis_index("s")
    base = (s * _SC_NC + c) * _SC_CHUNK
    for j in range(_SC_CHUNK // 16):
        p = base + j * 16 + lax.iota(jnp.int32, 16)
        d = p - (N - 1)
        nneg = -d
        na = jnp.abs(nneg)
        one = jnp.full((16,), 1, jnp.int32)
        zero = jnp.full((16,), 0, jnp.int32)
        large = jnp.full((16,), 8, jnp.int32)
        for t in _THRESHOLDS:
            large = large + jnp.where(na >= t, one, zero)
        bucket = jnp.where(na < 8, na, large) + jnp.where(
            nneg < 0, jnp.full((16,), 16, jnp.int32), zero
        )
        idx_v[pl.ds(j * 16, 16)] = bucket
    pltpu.async_copy(tableP_hbm.at[idx_v], rows_v, sem).wait()
    pltpu.sync_copy(rows_v, diagT_hbm.at[pl.ds(base, _SC_CHUNK), :])


_diag_sc = pl.kernel(
    _diag_sc_body,
    out_type=jax.ShapeDtypeStruct((D, HP), jnp.float32),
    mesh=plsc.VectorSubcoreMesh(core_axis_name="c", subcore_axis_name="s"),
    scratch_types=[
        pltpu.VMEM((_SC_CHUNK,), jnp.int32),
        pltpu.VMEM((_SC_CHUNK, HP), jnp.float32),
        pltpu.SemaphoreType.DMA,
    ],
    compiler_params=pltpu.CompilerParams(use_tc_tiling_on_sc=False),
)


def _expand_tc_body(diagT_ref, out_ref, rot_ref, diag_all_ref):
    h = pl.program_id(0)
    ib = pl.program_id(1)

    @pl.when((h == 0) & (ib == 0))
    def _transpose_and_mask():
        dall = jnp.transpose(diagT_ref[...], (1, 0))
        p = jax.lax.broadcasted_iota(jnp.int32, (1, D), 1)
        d = p - (N - 1)
        mask = -(((d > 32) | (d < -32)).astype(jnp.float32) * 100000000.0)
        diag_all_ref[...] = dall + jnp.broadcast_to(mask, (HP, D))

    @pl.when(ib == 0)
    def _build_rotations():
        bc = jnp.broadcast_to(diag_all_ref[pl.ds(h, 1), :], (128, D))
        rot_ref[...] = pltpu.roll(bc, D - 127, 1, stride=1, stride_axis=0)

    for g in range(BI // 128):
        a = pl.multiple_of((N - 1 - 127) - (ib * BI + g * 128), 128)
        out_ref[0, g * 128:(g + 1) * 128, :] = rot_ref[:, pl.ds(a, N)]


@jax.jit
def _bias_impl(table):
    tableP = jnp.pad(table, ((0, 0), (0, HP - H)))
    diagT = _diag_sc(tableP)
    return pl.pallas_call(
        _expand_tc_body,
        grid=(H, N // BI),
        in_specs=[pl.BlockSpec((D, HP), lambda h, ib: (0, 0))],
        out_specs=pl.BlockSpec((1, BI, N), lambda h, ib: (h, ib, 0)),
        out_shape=jax.ShapeDtypeStruct((H, N, N), jnp.float32),
        scratch_shapes=[
            pltpu.VMEM((128, D), jnp.float32),
            pltpu.VMEM((HP, D), jnp.float32),
        ],
        compiler_params=pltpu.CompilerParams(
            dimension_semantics=("arbitrary", "arbitrary"),
        ),
    )(diagT)


def kernel(n, table):
    del n
    return _bias_impl(table)
